# baseline (speedup 1.0000x reference)
import functools

import jax
import jax.numpy as jnp
from jax import lax
from jax.experimental import pallas as pl
from jax.experimental.pallas import tpu as pltpu
from jax.experimental.pallas import tpu_sc as plsc

N_NODES = 10000
N_PAD = 10240
N_EDGES = 320000
NC, NS = 2, 16
NW = NC * NS
AGG_C = 320
AGG_NCH = 32
EPWP = AGG_C * AGG_NCH
E_PAD = NW * EPWP - N_EDGES
RPT = N_PAD // NS
ROWBLK = 400
NBLK = N_NODES // ROWBLK


def _sc_mesh():
    return plsc.VectorSubcoreMesh(
        core_axis_name="c", subcore_axis_name="s", num_cores=NC, num_subcores=NS
    )


def _make_agg(F):
    C, NCH = AGG_C, AGG_NCH
    n_pairs = NCH // 2

    @functools.partial(
        pl.kernel,
        out_type=jax.ShapeDtypeStruct((NC, N_PAD, F), jnp.float32),
        mesh=_sc_mesh(),
        scratch_types=[
            pltpu.VMEM((C,), jnp.int32),
            pltpu.VMEM((C,), jnp.int32),
            pltpu.VMEM((C,), jnp.int32),
            pltpu.VMEM((C,), jnp.int32),
            pltpu.VMEM((C, F), jnp.float32),
            pltpu.VMEM_SHARED((N_PAD, F), jnp.float32),
            pltpu.SemaphoreType.DMA,
            pltpu.SemaphoreType.DMA,
            pltpu.SemaphoreType.DMA,
        ],
    )
    def agg(table_hbm, src_hbm, dst_hbm, zeros_hbm, out_hbm,
            sv0, dv0, sv1, dv1, rb, acc, si0, si1, sg):
        c = lax.axis_index("c")
        s = lax.axis_index("s")
        wid = c * NS + s
        base = wid * EPWP
        pltpu.sync_copy(zeros_hbm.at[pl.ds(s * RPT, RPT)],
                        acc.at[pl.ds(s * RPT, RPT)])
        plsc.subcore_barrier()

        def fire_idx(k, sv, dv, sem):
            pltpu.async_copy(src_hbm.at[pl.ds(base + k * C, C)], sv, sem)
            pltpu.async_copy(dst_hbm.at[pl.ds(base + k * C, C)], dv, sem)

        def wait_idx(k, sv, dv, sem):
            pltpu.make_async_copy(src_hbm.at[pl.ds(base + k * C, C)], sv,
                                  sem).wait()
            pltpu.make_async_copy(dst_hbm.at[pl.ds(base + k * C, C)], dv,
                                  sem).wait()

        def do_chunk(sv, dv):
            pltpu.async_copy(table_hbm.at[sv], rb, sg).wait()
            pltpu.sync_copy(rb, acc.at[dv], add=True)

        fire_idx(0, sv0, dv0, si0)

        def body(j, carry):
            k0 = 2 * j
            k1 = k0 + 1
            fire_idx(k1, sv1, dv1, si1)
            wait_idx(k0, sv0, dv0, si0)
            do_chunk(sv0, dv0)

            @pl.when(k0 + 2 < NCH)
            def _():
                fire_idx(k0 + 2, sv0, dv0, si0)

            wait_idx(k1, sv1, dv1, si1)
            do_chunk(sv1, dv1)
            return carry

        lax.fori_loop(0, n_pairs, body, 0)
        plsc.subcore_barrier()
        pltpu.sync_copy(acc.at[pl.ds(s * RPT, RPT)],
                        out_hbm.at[c, pl.ds(s * RPT, RPT)])

    return agg


_agg128 = _make_agg(128)


DEGW = 128


@functools.partial(
    pl.kernel,
    out_type=jax.ShapeDtypeStruct((NC, N_PAD, DEGW), jnp.float32),
    mesh=_sc_mesh(),
    scratch_types=[
        pltpu.VMEM((AGG_C,), jnp.int32),
        pltpu.VMEM((AGG_C,), jnp.int32),
        pltpu.VMEM((AGG_C, DEGW), jnp.float32),
        pltpu.VMEM_SHARED((N_PAD, DEGW), jnp.float32),
        pltpu.SemaphoreType.DMA,
        pltpu.SemaphoreType.DMA,
    ],
)
def _sc_deg(dst_hbm, ones_hbm, zeros_hbm, out_hbm, dv0, dv1, ones_v, acc,
            si0, si1):
    c = lax.axis_index("c")
    s = lax.axis_index("s")
    wid = c * NS + s
    base = wid * EPWP
    C, NCH = AGG_C, AGG_NCH
    pltpu.sync_copy(ones_hbm, ones_v)
    pltpu.sync_copy(zeros_hbm.at[pl.ds(s * RPT, RPT)],
                    acc.at[pl.ds(s * RPT, RPT)])
    plsc.subcore_barrier()

    def fire_idx(k, dv, sem):
        pltpu.async_copy(dst_hbm.at[pl.ds(base + k * C, C)], dv, sem)

    def wait_idx(k, dv, sem):
        pltpu.make_async_copy(dst_hbm.at[pl.ds(base + k * C, C)], dv,
                              sem).wait()

    fire_idx(0, dv0, si0)

    def body(j, carry):
        k0 = 2 * j
        k1 = k0 + 1
        fire_idx(k1, dv1, si1)
        wait_idx(k0, dv0, si0)
        pltpu.sync_copy(ones_v, acc.at[dv0], add=True)

        @pl.when(k0 + 2 < NCH)
        def _():
            fire_idx(k0 + 2, dv0, si0)

        wait_idx(k1, dv1, si1)
        pltpu.sync_copy(ones_v, acc.at[dv1], add=True)
        return carry

    lax.fori_loop(0, NCH // 2, body, 0)
    plsc.subcore_barrier()
    pltpu.sync_copy(acc.at[pl.ds(s * RPT, RPT)],
                    out_hbm.at[c, pl.ds(s * RPT, RPT)])


def _rowspec(F):
    return pl.BlockSpec((ROWBLK, F), lambda i: (i, 0))


def _partspec(F):
    return pl.BlockSpec((NC, ROWBLK, F), lambda i: (0, i, 0))


def _fullspec(shape):
    nd = len(shape)
    return pl.BlockSpec(shape, lambda i: (0,) * nd)


def _pre_body(degp_ref, x_ref, dis_ref, s0_ref):
    deg = degp_ref[0] + degp_ref[1] + 1.0
    dis = lax.rsqrt(deg)
    dis_ref[...] = dis
    s0_ref[...] = dis * x_ref[...]


_tc_pre = pl.pallas_call(
    _pre_body,
    grid=(NBLK,),
    in_specs=[_partspec(1), _rowspec(128)],
    out_specs=[_rowspec(1), _rowspec(128)],
    out_shape=[
        jax.ShapeDtypeStruct((N_NODES, 1), jnp.float32),
        jax.ShapeDtypeStruct((N_NODES, 128), jnp.float32),
    ],
)


def _mm12_body(p1_ref, s0_ref, dis_ref, w1_ref, b1_ref, w2_ref, zs2_ref):
    dis = dis_ref[...]
    t = dis * (p1_ref[0] + p1_ref[1] + s0_ref[...])
    h1 = jnp.maximum(
        jnp.dot(t, w1_ref[...], preferred_element_type=jnp.float32)
        + b1_ref[...], 0.0)
    z = dis * jnp.dot(h1, w2_ref[...], preferred_element_type=jnp.float32)
    zs2_ref[0] = z[:, :128]
    zs2_ref[1] = z[:, 128:]


_tc_mm12 = pl.pallas_call(
    _mm12_body,
    grid=(NBLK,),
    in_specs=[
        _partspec(128), _rowspec(128), _rowspec(1),
        _fullspec((128, 256)), _fullspec((1, 256)), _fullspec((256, 256)),
    ],
    out_specs=_partspec(128),
    out_shape=jax.ShapeDtypeStruct((2, N_NODES, 128), jnp.float32),
)


def _post2mm3_body(p2a_ref, p2b_ref, zs2_ref, dis_ref, b2_ref, w3_ref,
                   zs3_ref):
    dis = dis_ref[...]
    h2lo = jnp.maximum(dis * (p2a_ref[0] + p2a_ref[1] + zs2_ref[0])
                       + b2_ref[0], 0.0)
    h2hi = jnp.maximum(dis * (p2b_ref[0] + p2b_ref[1] + zs2_ref[1])
                       + b2_ref[1], 0.0)
    h2 = jnp.concatenate([h2lo, h2hi], axis=1)
    zs3_ref[...] = dis * jnp.dot(h2, w3_ref[...],
                                 preferred_element_type=jnp.float32)


_tc_post2mm3 = pl.pallas_call(
    _post2mm3_body,
    grid=(NBLK,),
    in_specs=[
        _partspec(128), _partspec(128), _partspec(128), _rowspec(1),
        _fullspec((2, 1, 128)), _fullspec((256, 128)),
    ],
    out_specs=_rowspec(128),
    out_shape=jax.ShapeDtypeStruct((N_NODES, 128), jnp.float32),
)


def _post3_body(p3_ref, zs3_ref, dis_ref, b3_ref, out_ref):
    out_ref[...] = (dis_ref[...] * (p3_ref[0] + p3_ref[1] + zs3_ref[...])
                    + b3_ref[...])


_tc_post3 = pl.pallas_call(
    _post3_body,
    grid=(NBLK,),
    in_specs=[_partspec(128), _rowspec(128), _rowspec(1), _fullspec((1, 128))],
    out_specs=_rowspec(128),
    out_shape=jax.ShapeDtypeStruct((N_NODES, 128), jnp.float32),
)


def kernel(x, edge_index, W1, b1, W2, b2, W3, b3):
    src = jnp.concatenate(
        [edge_index[0].astype(jnp.int32), jnp.zeros((E_PAD,), jnp.int32)])
    pad_rows = N_NODES + (jnp.arange(E_PAD, dtype=jnp.int32)
                          % (N_PAD - N_NODES))
    dst = jnp.concatenate([edge_index[1].astype(jnp.int32), pad_rows])

    zeros128 = jnp.zeros((N_PAD, 128), jnp.float32)
    zerosw = jnp.zeros((N_PAD, DEGW), jnp.float32)
    onesw = jnp.ones((AGG_C, DEGW), jnp.float32)

    degp = _sc_deg(dst, onesw, zerosw)
    dis, s0 = _tc_pre(degp[:, :N_NODES, 0:1], x)

    def agg(t, F):
        return _agg128(t, src, dst, zeros128)[:, :N_NODES]

    P1 = agg(s0, 128)
    zs2 = _tc_mm12(P1, s0, dis, W1, b1.reshape(1, 256), W2)

    P2a = agg(zs2[0], 128)
    P2b = agg(zs2[1], 128)
    W3p = jnp.pad(W3, ((0, 0), (0, 88)))
    zs3 = _tc_post2mm3(P2a, P2b, zs2, dis, b2.reshape(2, 1, 128), W3p)

    P3 = agg(zs3, 128)
    b3p = jnp.pad(b3, (0, 88)).reshape(1, 128)
    out = _tc_post3(P3, zs3, dis, b3p)
    return out[:, :40]


# speedup vs baseline: 2.3617x; 2.3617x over previous
import functools

import jax
import jax.numpy as jnp
from jax import lax
from jax.experimental import pallas as pl
from jax.experimental.pallas import tpu as pltpu
from jax.experimental.pallas import tpu_sc as plsc

N_NODES = 10000
N_PAD = 10240
N_EDGES = 320000
NC, NS = 2, 16
NW = NC * NS
AGG_C = 200
AGG_NCH = 50
EPWP = AGG_C * AGG_NCH
E_PAD = NW * EPWP - N_EDGES
RPT = N_PAD // NS
ROWBLK = 400
NBLK = N_NODES // ROWBLK


def _sc_mesh():
    return plsc.VectorSubcoreMesh(
        core_axis_name="c", subcore_axis_name="s", num_cores=NC, num_subcores=NS
    )


def _make_agg(F):
    C, NCH = AGG_C, AGG_NCH
    n_pairs = NCH // 2

    @functools.partial(
        pl.kernel,
        out_type=jax.ShapeDtypeStruct((NC, N_PAD, F), jnp.float32),
        mesh=_sc_mesh(),
        scratch_types=[
            pltpu.VMEM((C,), jnp.int32),
            pltpu.VMEM((C,), jnp.int32),
            pltpu.VMEM((C,), jnp.int32),
            pltpu.VMEM((C,), jnp.int32),
            pltpu.VMEM((C, F), jnp.float32),
            pltpu.VMEM_SHARED((N_PAD, F), jnp.float32),
            pltpu.SemaphoreType.DMA,
            pltpu.SemaphoreType.DMA,
            pltpu.SemaphoreType.DMA,
        ],
    )
    def agg(table_hbm, src_hbm, dst_hbm, zeros_hbm, out_hbm,
            sv0, dv0, sv1, dv1, rb, acc, si0, si1, sg):
        c = lax.axis_index("c")
        s = lax.axis_index("s")
        wid = c * NS + s
        base = wid * EPWP
        pltpu.sync_copy(zeros_hbm.at[pl.ds(s * RPT, RPT)],
                        acc.at[pl.ds(s * RPT, RPT)])
        plsc.subcore_barrier()

        def fire_idx(k, sv, dv, sem):
            pltpu.async_copy(src_hbm.at[pl.ds(base + k * C, C)], sv, sem)
            pltpu.async_copy(dst_hbm.at[pl.ds(base + k * C, C)], dv, sem)

        def wait_idx(k, sv, dv, sem):
            pltpu.make_async_copy(src_hbm.at[pl.ds(base + k * C, C)], sv,
                                  sem).wait()
            pltpu.make_async_copy(dst_hbm.at[pl.ds(base + k * C, C)], dv,
                                  sem).wait()

        def do_chunk(sv, dv):
            pltpu.async_copy(table_hbm.at[sv], rb, sg).wait()
            pltpu.sync_copy(rb, acc.at[dv], add=True)

        fire_idx(0, sv0, dv0, si0)

        def body(j, carry):
            k0 = 2 * j
            k1 = k0 + 1
            fire_idx(k1, sv1, dv1, si1)
            wait_idx(k0, sv0, dv0, si0)
            do_chunk(sv0, dv0)

            @pl.when(k0 + 2 < NCH)
            def _():
                fire_idx(k0 + 2, sv0, dv0, si0)

            wait_idx(k1, sv1, dv1, si1)
            do_chunk(sv1, dv1)
            return carry

        lax.fori_loop(0, n_pairs, body, 0)
        plsc.subcore_barrier()
        pltpu.sync_copy(acc.at[pl.ds(s * RPT, RPT)],
                        out_hbm.at[c, pl.ds(s * RPT, RPT)])

    return agg


_agg128 = _make_agg(128)


DEGW = 128


@functools.partial(
    pl.kernel,
    out_type=jax.ShapeDtypeStruct((NC, N_PAD, DEGW), jnp.float32),
    mesh=_sc_mesh(),
    scratch_types=[
        pltpu.VMEM((AGG_C,), jnp.int32),
        pltpu.VMEM((AGG_C,), jnp.int32),
        pltpu.VMEM((AGG_C, DEGW), jnp.float32),
        pltpu.VMEM_SHARED((N_PAD, DEGW), jnp.float32),
        pltpu.SemaphoreType.DMA,
        pltpu.SemaphoreType.DMA,
    ],
)
def _sc_deg(dst_hbm, ones_hbm, zeros_hbm, out_hbm, dv0, dv1, ones_v, acc,
            si0, si1):
    c = lax.axis_index("c")
    s = lax.axis_index("s")
    wid = c * NS + s
    base = wid * EPWP
    C, NCH = AGG_C, AGG_NCH
    pltpu.sync_copy(ones_hbm, ones_v)
    pltpu.sync_copy(zeros_hbm.at[pl.ds(s * RPT, RPT)],
                    acc.at[pl.ds(s * RPT, RPT)])
    plsc.subcore_barrier()

    def fire_idx(k, dv, sem):
        pltpu.async_copy(dst_hbm.at[pl.ds(base + k * C, C)], dv, sem)

    def wait_idx(k, dv, sem):
        pltpu.make_async_copy(dst_hbm.at[pl.ds(base + k * C, C)], dv,
                              sem).wait()

    fire_idx(0, dv0, si0)

    def body(j, carry):
        k0 = 2 * j
        k1 = k0 + 1
        fire_idx(k1, dv1, si1)
        wait_idx(k0, dv0, si0)
        pltpu.sync_copy(ones_v, acc.at[dv0], add=True)

        @pl.when(k0 + 2 < NCH)
        def _():
            fire_idx(k0 + 2, dv0, si0)

        wait_idx(k1, dv1, si1)
        pltpu.sync_copy(ones_v, acc.at[dv1], add=True)
        return carry

    lax.fori_loop(0, NCH // 2, body, 0)
    plsc.subcore_barrier()
    pltpu.sync_copy(acc.at[pl.ds(s * RPT, RPT)],
                    out_hbm.at[c, pl.ds(s * RPT, RPT)])


def _rowspec(F):
    return pl.BlockSpec((ROWBLK, F), lambda i: (i, 0))


def _partspec(F):
    return pl.BlockSpec((NC, ROWBLK, F), lambda i: (0, i, 0))


def _fullspec(shape):
    nd = len(shape)
    return pl.BlockSpec(shape, lambda i: (0,) * nd)


def _pre_body(degp_ref, x_ref, dis_ref, s0_ref):
    deg = degp_ref[0] + degp_ref[1] + 1.0
    dis = lax.rsqrt(deg)
    dis_ref[...] = dis
    s0_ref[...] = dis * x_ref[...]


_tc_pre = pl.pallas_call(
    _pre_body,
    grid=(NBLK,),
    in_specs=[_partspec(1), _rowspec(128)],
    out_specs=[_rowspec(1), _rowspec(128)],
    out_shape=[
        jax.ShapeDtypeStruct((N_NODES, 1), jnp.float32),
        jax.ShapeDtypeStruct((N_NODES, 128), jnp.float32),
    ],
)


def _mm12_body(p1_ref, s0_ref, dis_ref, w1_ref, b1_ref, w2_ref, zs2_ref):
    dis = dis_ref[...]
    t = dis * (p1_ref[0] + p1_ref[1] + s0_ref[...])
    h1 = jnp.maximum(
        jnp.dot(t, w1_ref[...], preferred_element_type=jnp.float32)
        + b1_ref[...], 0.0)
    z = dis * jnp.dot(h1, w2_ref[...], preferred_element_type=jnp.float32)
    zs2_ref[0] = z[:, :128]
    zs2_ref[1] = z[:, 128:]


_tc_mm12 = pl.pallas_call(
    _mm12_body,
    grid=(NBLK,),
    in_specs=[
        _partspec(128), _rowspec(128), _rowspec(1),
        _fullspec((128, 256)), _fullspec((1, 256)), _fullspec((256, 256)),
    ],
    out_specs=_partspec(128),
    out_shape=jax.ShapeDtypeStruct((2, N_NODES, 128), jnp.float32),
)


def _post2mm3_body(p2a_ref, p2b_ref, zs2_ref, dis_ref, b2_ref, w3_ref,
                   zs3_ref):
    dis = dis_ref[...]
    h2lo = jnp.maximum(dis * (p2a_ref[0] + p2a_ref[1] + zs2_ref[0])
                       + b2_ref[0], 0.0)
    h2hi = jnp.maximum(dis * (p2b_ref[0] + p2b_ref[1] + zs2_ref[1])
                       + b2_ref[1], 0.0)
    h2 = jnp.concatenate([h2lo, h2hi], axis=1)
    zs3_ref[...] = dis * jnp.dot(h2, w3_ref[...],
                                 preferred_element_type=jnp.float32)


_tc_post2mm3 = pl.pallas_call(
    _post2mm3_body,
    grid=(NBLK,),
    in_specs=[
        _partspec(128), _partspec(128), _partspec(128), _rowspec(1),
        _fullspec((2, 1, 128)), _fullspec((256, 128)),
    ],
    out_specs=_rowspec(128),
    out_shape=jax.ShapeDtypeStruct((N_NODES, 128), jnp.float32),
)


def _post3_body(p3_ref, zs3_ref, dis_ref, b3_ref, out_ref):
    out_ref[...] = (dis_ref[...] * (p3_ref[0] + p3_ref[1] + zs3_ref[...])
                    + b3_ref[...])


_tc_post3 = pl.pallas_call(
    _post3_body,
    grid=(NBLK,),
    in_specs=[_partspec(128), _rowspec(128), _rowspec(1), _fullspec((1, 128))],
    out_specs=_rowspec(128),
    out_shape=jax.ShapeDtypeStruct((N_NODES, 128), jnp.float32),
)


def kernel(x, edge_index, W1, b1, W2, b2, W3, b3):
    src = edge_index[0].astype(jnp.int32)
    dst = edge_index[1].astype(jnp.int32)

    zeros128 = jnp.zeros((N_PAD, 128), jnp.float32)
    zerosw = jnp.zeros((N_PAD, DEGW), jnp.float32)
    onesw = jnp.ones((AGG_C, DEGW), jnp.float32)

    degp = _sc_deg(dst, onesw, zerosw)
    dis, s0 = _tc_pre(degp[:, :N_NODES, 0:1], x)

    def agg(t, F):
        return _agg128(t, src, dst, zeros128)[:, :N_NODES]

    P1 = agg(s0, 128)
    zs2 = _tc_mm12(P1, s0, dis, W1, b1.reshape(1, 256), W2)

    P2a = agg(zs2[0], 128)
    P2b = agg(zs2[1], 128)
    W3p = jnp.pad(W3, ((0, 0), (0, 88)))
    zs3 = _tc_post2mm3(P2a, P2b, zs2, dis, b2.reshape(2, 1, 128), W3p)

    P3 = agg(zs3, 128)
    b3p = jnp.pad(b3, (0, 88)).reshape(1, 128)
    out = _tc_post3(P3, zs3, dis, b3p)
    return out[:, :40]


# speedup vs baseline: 2.3644x; 1.0011x over previous
import functools

import jax
import jax.numpy as jnp
from jax import lax
from jax.experimental import pallas as pl
from jax.experimental.pallas import tpu as pltpu
from jax.experimental.pallas import tpu_sc as plsc

N_NODES = 10000
N_PAD = 10240
N_EDGES = 320000
NC, NS = 2, 16
NW = NC * NS
AGG_C = 200
AGG_NCH = 50
EPWP = AGG_C * AGG_NCH
E_PAD = NW * EPWP - N_EDGES
RPT = N_PAD // NS
ROWBLK = 400
NBLK = N_NODES // ROWBLK


def _sc_mesh():
    return plsc.VectorSubcoreMesh(
        core_axis_name="c", subcore_axis_name="s", num_cores=NC, num_subcores=NS
    )


def _make_agg(F):
    C, NCH = AGG_C, AGG_NCH
    n_pairs = NCH // 2

    @functools.partial(
        pl.kernel,
        out_type=jax.ShapeDtypeStruct((NC, N_PAD, F), jnp.float32),
        mesh=_sc_mesh(),
        scratch_types=[
            pltpu.VMEM((C,), jnp.int32),
            pltpu.VMEM((C,), jnp.int32),
            pltpu.VMEM((C,), jnp.int32),
            pltpu.VMEM((C,), jnp.int32),
            pltpu.VMEM((C, F), jnp.float32),
            pltpu.VMEM_SHARED((N_PAD, F), jnp.float32),
            pltpu.SemaphoreType.DMA,
            pltpu.SemaphoreType.DMA,
            pltpu.SemaphoreType.DMA,
        ],
    )
    def agg(table_hbm, src_hbm, dst_hbm, zeros_hbm, out_hbm,
            sv0, dv0, sv1, dv1, rb, acc, si0, si1, sg):
        c = lax.axis_index("c")
        s = lax.axis_index("s")
        wid = c * NS + s
        base = wid * EPWP
        pltpu.sync_copy(zeros_hbm.at[pl.ds(s * RPT, RPT)],
                        acc.at[pl.ds(s * RPT, RPT)])
        plsc.subcore_barrier()

        def fire_idx(k, sv, dv, sem):
            pltpu.async_copy(src_hbm.at[pl.ds(base + k * C, C)], sv, sem)
            pltpu.async_copy(dst_hbm.at[pl.ds(base + k * C, C)], dv, sem)

        def wait_idx(k, sv, dv, sem):
            pltpu.make_async_copy(src_hbm.at[pl.ds(base + k * C, C)], sv,
                                  sem).wait()
            pltpu.make_async_copy(dst_hbm.at[pl.ds(base + k * C, C)], dv,
                                  sem).wait()

        def do_chunk(sv, dv):
            pltpu.async_copy(table_hbm.at[sv], rb, sg).wait()
            pltpu.sync_copy(rb, acc.at[dv], add=True)

        fire_idx(0, sv0, dv0, si0)

        def body(j, carry):
            k0 = 2 * j
            k1 = k0 + 1
            fire_idx(k1, sv1, dv1, si1)
            wait_idx(k0, sv0, dv0, si0)
            do_chunk(sv0, dv0)

            @pl.when(k0 + 2 < NCH)
            def _():
                fire_idx(k0 + 2, sv0, dv0, si0)

            wait_idx(k1, sv1, dv1, si1)
            do_chunk(sv1, dv1)
            return carry

        lax.fori_loop(0, n_pairs, body, 0)
        plsc.subcore_barrier()
        pltpu.sync_copy(acc.at[pl.ds(s * RPT, RPT)],
                        out_hbm.at[c, pl.ds(s * RPT, RPT)])

    return agg


_agg128 = _make_agg(128)


DEGW = 128


@functools.partial(
    pl.kernel,
    out_type=jax.ShapeDtypeStruct((NC, N_PAD, DEGW), jnp.float32),
    mesh=_sc_mesh(),
    scratch_types=[
        pltpu.VMEM((AGG_C,), jnp.int32),
        pltpu.VMEM((AGG_C,), jnp.int32),
        pltpu.VMEM((AGG_C, DEGW), jnp.float32),
        pltpu.VMEM_SHARED((N_PAD, DEGW), jnp.float32),
        pltpu.SemaphoreType.DMA,
        pltpu.SemaphoreType.DMA,
    ],
)
def _sc_deg(dst_hbm, ones_hbm, zeros_hbm, out_hbm, dv0, dv1, ones_v, acc,
            si0, si1):
    c = lax.axis_index("c")
    s = lax.axis_index("s")
    wid = c * NS + s
    base = wid * EPWP
    C, NCH = AGG_C, AGG_NCH
    pltpu.sync_copy(ones_hbm, ones_v)
    pltpu.sync_copy(zeros_hbm.at[pl.ds(s * RPT, RPT)],
                    acc.at[pl.ds(s * RPT, RPT)])
    plsc.subcore_barrier()

    def fire_idx(k, dv, sem):
        pltpu.async_copy(dst_hbm.at[pl.ds(base + k * C, C)], dv, sem)

    def wait_idx(k, dv, sem):
        pltpu.make_async_copy(dst_hbm.at[pl.ds(base + k * C, C)], dv,
                              sem).wait()

    fire_idx(0, dv0, si0)

    def body(j, carry):
        k0 = 2 * j
        k1 = k0 + 1
        fire_idx(k1, dv1, si1)
        wait_idx(k0, dv0, si0)
        pltpu.sync_copy(ones_v, acc.at[dv0], add=True)

        @pl.when(k0 + 2 < NCH)
        def _():
            fire_idx(k0 + 2, dv0, si0)

        wait_idx(k1, dv1, si1)
        pltpu.sync_copy(ones_v, acc.at[dv1], add=True)
        return carry

    lax.fori_loop(0, NCH // 2, body, 0)
    plsc.subcore_barrier()
    pltpu.sync_copy(acc.at[pl.ds(s * RPT, RPT)],
                    out_hbm.at[c, pl.ds(s * RPT, RPT)])


def _rowspec(F):
    return pl.BlockSpec((ROWBLK, F), lambda i: (i, 0))


def _partspec(F):
    return pl.BlockSpec((NC, ROWBLK, F), lambda i: (0, i, 0))


def _fullspec(shape):
    nd = len(shape)
    return pl.BlockSpec(shape, lambda i: (0,) * nd)


def _pre_body(degp_ref, x_ref, dis_ref, s0_ref):
    deg = degp_ref[0] + degp_ref[1] + 1.0
    dis = lax.rsqrt(deg)
    dis_ref[...] = dis
    s0_ref[...] = dis * x_ref[...]


_tc_pre = pl.pallas_call(
    _pre_body,
    grid=(NBLK,),
    in_specs=[_partspec(1), _rowspec(128)],
    out_specs=[_rowspec(1), _rowspec(128)],
    out_shape=[
        jax.ShapeDtypeStruct((N_NODES, 1), jnp.float32),
        jax.ShapeDtypeStruct((N_NODES, 128), jnp.float32),
    ],
)


def _mm12_body(p1_ref, s0_ref, dis_ref, w1_ref, b1_ref, w2_ref, zs2_ref):
    dis = dis_ref[...]
    t = dis * (p1_ref[0] + p1_ref[1] + s0_ref[...])
    h1 = jnp.maximum(
        jnp.dot(t.astype(jnp.bfloat16), w1_ref[...].astype(jnp.bfloat16),
                preferred_element_type=jnp.float32)
        + b1_ref[...], 0.0)
    z = dis * jnp.dot(h1.astype(jnp.bfloat16),
                      w2_ref[...].astype(jnp.bfloat16),
                      preferred_element_type=jnp.float32)
    zs2_ref[0] = z[:, :128]
    zs2_ref[1] = z[:, 128:]


_tc_mm12 = pl.pallas_call(
    _mm12_body,
    grid=(NBLK,),
    in_specs=[
        _partspec(128), _rowspec(128), _rowspec(1),
        _fullspec((128, 256)), _fullspec((1, 256)), _fullspec((256, 256)),
    ],
    out_specs=_partspec(128),
    out_shape=jax.ShapeDtypeStruct((2, N_NODES, 128), jnp.float32),
)


def _post2mm3_body(p2a_ref, p2b_ref, zs2_ref, dis_ref, b2_ref, w3_ref,
                   zs3_ref):
    dis = dis_ref[...]
    h2lo = jnp.maximum(dis * (p2a_ref[0] + p2a_ref[1] + zs2_ref[0])
                       + b2_ref[0], 0.0)
    h2hi = jnp.maximum(dis * (p2b_ref[0] + p2b_ref[1] + zs2_ref[1])
                       + b2_ref[1], 0.0)
    h2 = jnp.concatenate([h2lo, h2hi], axis=1)
    zs3_ref[...] = dis * jnp.dot(h2.astype(jnp.bfloat16),
                                 w3_ref[...].astype(jnp.bfloat16),
                                 preferred_element_type=jnp.float32)


_tc_post2mm3 = pl.pallas_call(
    _post2mm3_body,
    grid=(NBLK,),
    in_specs=[
        _partspec(128), _partspec(128), _partspec(128), _rowspec(1),
        _fullspec((2, 1, 128)), _fullspec((256, 128)),
    ],
    out_specs=_rowspec(128),
    out_shape=jax.ShapeDtypeStruct((N_NODES, 128), jnp.float32),
)


def _post3_body(p3_ref, zs3_ref, dis_ref, b3_ref, out_ref):
    out_ref[...] = (dis_ref[...] * (p3_ref[0] + p3_ref[1] + zs3_ref[...])
                    + b3_ref[...])


_tc_post3 = pl.pallas_call(
    _post3_body,
    grid=(NBLK,),
    in_specs=[_partspec(128), _rowspec(128), _rowspec(1), _fullspec((1, 128))],
    out_specs=_rowspec(128),
    out_shape=jax.ShapeDtypeStruct((N_NODES, 128), jnp.float32),
)


def kernel(x, edge_index, W1, b1, W2, b2, W3, b3):
    src = edge_index[0].astype(jnp.int32)
    dst = edge_index[1].astype(jnp.int32)

    zeros128 = jnp.zeros((N_PAD, 128), jnp.float32)
    zerosw = jnp.zeros((N_PAD, DEGW), jnp.float32)
    onesw = jnp.ones((AGG_C, DEGW), jnp.float32)

    degp = _sc_deg(dst, onesw, zerosw)
    dis, s0 = _tc_pre(degp[:, :N_NODES, 0:1], x)

    def agg(t, F):
        return _agg128(t, src, dst, zeros128)[:, :N_NODES]

    P1 = agg(s0, 128)
    zs2 = _tc_mm12(P1, s0, dis, W1, b1.reshape(1, 256), W2)

    P2a = agg(zs2[0], 128)
    P2b = agg(zs2[1], 128)
    W3p = jnp.pad(W3, ((0, 0), (0, 88)))
    zs3 = _tc_post2mm3(P2a, P2b, zs2, dis, b2.reshape(2, 1, 128), W3p)

    P3 = agg(zs3, 128)
    b3p = jnp.pad(b3, (0, 88)).reshape(1, 128)
    out = _tc_post3(P3, zs3, dis, b3p)
    return out[:, :40]
